# baseline (speedup 1.0000x reference)
import functools

import jax
import jax.numpy as jnp
from jax import lax
from jax.experimental import pallas as pl
from jax.experimental.pallas import tpu as pltpu
from jax.experimental.pallas import tpu_sc as plsc

_B = 4096
_L = 200
_V = 100000
_EMBED = 64
_N = _B * _L
_NC = 2
_NS = 16
_NW = _NC * _NS
_PER_W = _N // _NW
_W = 128
_T = _PER_W // _W
_WPB = 2
_NBUF = 2
_PADDED = 128


def kernel(indices, table):
    flat_idx = indices.reshape(_N).astype(jnp.int32)
    padded = jnp.pad(table, ((0, 0), (0, _PADDED - _EMBED)))

    mesh = plsc.VectorSubcoreMesh(core_axis_name="c", subcore_axis_name="s")

    @functools.partial(
        pl.kernel,
        out_type=jax.ShapeDtypeStruct((_N, _PADDED), jnp.float32),
        mesh=mesh,
        scratch_types=[
            pltpu.VMEM((_PER_W,), jnp.int32),
            *[pltpu.VMEM((_W * _WPB, _PADDED), jnp.float32) for _ in range(_NBUF)],
            *[pltpu.SemaphoreType.DMA for _ in range(_NBUF * _WPB + _NBUF)],
        ],
    )
    def gather_kernel(table_hbm, idx_hbm, out_hbm, idx_all, *scratch):
        rows = scratch[:_NBUF]
        gsem = scratch[_NBUF:_NBUF + _NBUF * _WPB]
        wsem = scratch[_NBUF + _NBUF * _WPB:]

        wid = lax.axis_index("s") * _NC + lax.axis_index("c")
        base = wid * _PER_W

        pltpu.sync_copy(idx_hbm.at[pl.ds(base, _PER_W)], idx_all)

        def gather_start(w, b, h):
            pltpu.async_copy(
                table_hbm.at[idx_all.at[pl.ds(w * _W, _W)]],
                rows[b].at[pl.ds(h * _W, _W)],
                gsem[b * _WPB + h],
            )

        def gather_wait(b, h):
            pltpu.make_async_copy(
                table_hbm.at[idx_all.at[pl.ds(0, _W)]],
                rows[b].at[pl.ds(h * _W, _W)],
                gsem[b * _WPB + h],
            ).wait()

        def write_start(w, b):
            pltpu.async_copy(
                rows[b], out_hbm.at[pl.ds(base + w * _W, _W * _WPB)], wsem[b]
            )

        def write_wait(b):
            pltpu.make_async_copy(
                rows[b], out_hbm.at[pl.ds(base, _W * _WPB)], wsem[b]
            ).wait()

        for b in range(_NBUF):
            for h in range(_WPB):
                gather_start(b * _WPB + h, b, h)

        @pl.loop(0, _T, step=_NBUF * _WPB)
        def _(g):
            for b in range(_NBUF):
                for h in range(_WPB):
                    gather_wait(b, h)
                write_start(g + b * _WPB, b)
            for b in range(_NBUF):
                write_wait(b)
                for h in range(_WPB):

                    @pl.when(g + (b + _NBUF) * _WPB + h < _T)
                    def _():
                        gather_start(g + (b + _NBUF) * _WPB + h, b, h)

    out = gather_kernel(padded, flat_idx)
    return out[:, :_EMBED].reshape(_B, _L, _EMBED)


# speedup vs baseline: 1.0054x; 1.0054x over previous
import functools

import jax
import jax.numpy as jnp
from jax import lax
from jax.experimental import pallas as pl
from jax.experimental.pallas import tpu as pltpu
from jax.experimental.pallas import tpu_sc as plsc

_B = 4096
_L = 200
_V = 100000
_EMBED = 64
_N = _B * _L
_NC = 2
_NS = 16
_NW = _NC * _NS
_PER_W = _N // _NW
_W = 128
_T = _PER_W // _W
_NBUF = 4
_PADDED = 128


def kernel(indices, table):
    flat_idx = indices.reshape(_N).astype(jnp.int32)
    padded = jnp.pad(table, ((0, 0), (0, _PADDED - _EMBED)))

    mesh = plsc.VectorSubcoreMesh(core_axis_name="c", subcore_axis_name="s")

    @functools.partial(
        pl.kernel,
        out_type=jax.ShapeDtypeStruct((_N, _PADDED), jnp.float32),
        mesh=mesh,
        scratch_types=[
            pltpu.VMEM((_PER_W,), jnp.int32),
            *[pltpu.VMEM((_W, _PADDED), jnp.float32) for _ in range(_NBUF)],
            *[pltpu.SemaphoreType.DMA for _ in range(2 * _NBUF)],
        ],
    )
    def gather_kernel(table_hbm, idx_hbm, out_hbm, idx_all, *scratch):
        rows = scratch[:_NBUF]
        gsem = scratch[_NBUF:2 * _NBUF]
        wsem = scratch[2 * _NBUF:]

        wid = lax.axis_index("s") * _NC + lax.axis_index("c")
        base = wid * _PER_W

        pltpu.sync_copy(idx_hbm.at[pl.ds(base, _PER_W)], idx_all)

        def gather_start(w, b):
            pltpu.async_copy(
                table_hbm.at[idx_all.at[pl.ds(w * _W, _W)]], rows[b], gsem[b]
            )

        def gather_wait(b):
            pltpu.make_async_copy(
                table_hbm.at[idx_all.at[pl.ds(0, _W)]], rows[b], gsem[b]
            ).wait()

        def write_start(w, b):
            pltpu.async_copy(rows[b], out_hbm.at[pl.ds(base + w * _W, _W)], wsem[b])

        def write_wait(b):
            pltpu.make_async_copy(
                rows[b], out_hbm.at[pl.ds(base, _W)], wsem[b]
            ).wait()

        for b in range(_NBUF):
            gather_start(b, b)

        @pl.loop(0, _T, step=_NBUF)
        def _(g):
            for b in range(_NBUF):
                gather_wait(b)
                write_start(g + b, b)
            for b in range(_NBUF):
                write_wait(b)

                @pl.when(g + b + _NBUF < _T)
                def _():
                    gather_start(g + b + _NBUF, b)

    out = gather_kernel(padded, flat_idx)
    return out[:, :_EMBED].reshape(_B, _L, _EMBED)
